# baseline (speedup 1.0000x reference)
import functools

import jax
import jax.numpy as jnp
from jax.experimental import pallas as pl

_QB = 512
_PAD = 16


def _th_kernel(q_ref, kt_ref, k16_ref, out_ref):
    d = pl.program_id(0)
    q = q_ref[0, 0]
    kt = kt_ref[0, 0]
    qx, qy, qz = q[:, 0:1], q[:, 1:2], q[:, 2:3]
    kx, ky, kz = kt[0:1, :], kt[1:2, :], kt[2:3, :]
    qq = qx * qx + qy * qy + qz * qz
    kk = kx * kx + ky * ky + kz * kz
    qk = jnp.dot(q.astype(jnp.bfloat16), kt.astype(jnp.bfloat16),
                 preferred_element_type=jnp.float32)
    d2 = (qq - 2.0 * qk) + kk
    ai = jnp.argmin(d2, axis=1, keepdims=True)
    oh = (ai == jax.lax.broadcasted_iota(jnp.int32, d2.shape, 1)).astype(
        jnp.float32)
    g = jnp.dot(oh, k16_ref[0, 0], preferred_element_type=jnp.float32,
                precision=jax.lax.Precision.HIGHEST)
    sign = (1 - 2 * (d % 2)).astype(jnp.float32)
    out_ref[0, 0] = sign * (g - q)


def _loss_kernel(blk_ref, full_ref, idx_ref, out_ref, *, nb, nbatch, scale):
    p, s, b, i = (pl.program_id(k) for k in range(4))

    @pl.when((p == 0) & (s == 0) & (b == 0) & (i == 0))
    def _init():
        out_ref[:, :] = jnp.zeros((1, 1), jnp.float32)

    idxc = idx_ref[0, 0, 0]
    full = full_ref[0, 0]
    oh = (idxc == jax.lax.broadcasted_iota(
        jnp.int32, (idxc.shape[0], full.shape[0]), 1)).astype(jnp.float32)
    g = jnp.dot(oh, full, preferred_element_type=jnp.float32,
                precision=jax.lax.Precision.HIGHEST)
    diff = blk_ref[0, 0] - g
    out_ref[:, :] += jnp.sum(diff * diff)[None, None]

    @pl.when((p == 1) & (s == 1) & (b == nbatch - 1) & (i == nb - 1))
    def _fin():
        out_ref[:, :] = out_ref[:, :] * scale


def kernel(yp_white_pts, yp_pial_pts, yt_white_pts, yt_pial_pts,
           yp_white_idx, yt_white_idx, yp_pial_idx, yt_pial_idx):
    B, N, _ = yp_white_pts.shape
    nb = N // _QB

    P = jnp.stack([yp_white_pts, yp_pial_pts, yt_white_pts, yt_pial_pts])
    P16 = jnp.pad(P, ((0, 0), (0, 0), (0, 0), (0, _PAD - 3)))
    PT = jnp.swapaxes(P16, 2, 3)

    sw = lambda d: (d // 2) * 2 + (1 - d % 2)

    th16 = pl.pallas_call(
        _th_kernel,
        grid=(4, B, nb),
        in_specs=[
            pl.BlockSpec((1, 1, _QB, _PAD), lambda d, b, i: (d, b, i, 0)),
            pl.BlockSpec((1, 1, _PAD, N), lambda d, b, i: (sw(d), b, 0, 0)),
            pl.BlockSpec((1, 1, N, _PAD), lambda d, b, i: (sw(d), b, 0, 0)),
        ],
        out_specs=pl.BlockSpec((1, 1, _QB, _PAD), lambda d, b, i: (d, b, i, 0)),
        out_shape=jax.ShapeDtypeStruct((4, B, N, _PAD), jnp.float32),
    )(P16, PT, P16)

    I = jnp.stack([yp_white_idx, yp_pial_idx, yt_white_idx, yt_pial_idx])
    I = I.astype(jnp.int32).reshape(4, B, nb, _QB, 1)

    loss = pl.pallas_call(
        functools.partial(_loss_kernel, nb=nb, nbatch=B,
                          scale=0.25 / (B * N)),
        grid=(2, 2, B, nb),
        in_specs=[
            pl.BlockSpec((1, 1, _QB, _PAD),
                         lambda p, s, b, i: (p + 2 * s, b, i, 0)),
            pl.BlockSpec((1, 1, N, _PAD),
                         lambda p, s, b, i: (p + 2 * (1 - s), b, 0, 0)),
            pl.BlockSpec((1, 1, 1, _QB, 1),
                         lambda p, s, b, i: (p + 2 * s, b, i, 0, 0)),
        ],
        out_specs=pl.BlockSpec((1, 1), lambda p, s, b, i: (0, 0)),
        out_shape=jax.ShapeDtypeStruct((1, 1), jnp.float32),
    )(th16, th16, I)

    return loss.reshape(())


# speedup vs baseline: 4.0303x; 4.0303x over previous
import functools

import jax
import jax.numpy as jnp
from jax import lax
from jax.experimental import pallas as pl
from jax.experimental.pallas import tpu as pltpu
from jax.experimental.pallas import tpu_sc as plsc

_QB = 512
_PAD = 16
_W = 128


def _th_kernel(q_ref, kt_ref, k16_ref, out_ref):
    d = pl.program_id(0)
    sign = (1 - 2 * (d % 2)).astype(jnp.float32)
    kt = kt_ref[0, 0]
    kx, ky, kz = kt[0:1, :], kt[1:2, :], kt[2:3, :]
    kk = kx * kx + ky * ky + kz * kz
    ktd_bf = (kt + kt).astype(jnp.bfloat16)
    k16_bf = k16_ref[0, 0].astype(jnp.bfloat16)
    h = _QB // 2
    for j in range(2):
        q = q_ref[0, 0][j * h:(j + 1) * h]
        qk2 = jnp.dot(q.astype(jnp.bfloat16), ktd_bf,
                      preferred_element_type=jnp.float32)
        score = kk - qk2
        m = jnp.min(score, axis=1, keepdims=True)
        oh = (score == m).astype(jnp.bfloat16)
        g = jnp.dot(oh, k16_bf, preferred_element_type=jnp.float32)
        g = g * (1.0 / g[:, _PAD - 1:_PAD])
        th = sign * (g - q)
        out_ref[0, 0, j * h:(j + 1) * h] = jnp.concatenate(
            [th, jnp.zeros((h, _W - _PAD), jnp.float32)], axis=1)


def _reduce_kernel(blk_ref, lg_ref, out_ref, *, nbatch, scale):
    p, s, b = (pl.program_id(k) for k in range(3))

    @pl.when((p == 0) & (s == 0) & (b == 0))
    def _init():
        out_ref[:, :] = jnp.zeros((1, 1), jnp.float32)

    diff = blk_ref[0, 0] - lg_ref[0, 0, 0]
    out_ref[:, :] += jnp.sum(diff * diff)[None, None]

    @pl.when((p == 1) & (s == 1) & (b == nbatch - 1))
    def _fin():
        out_ref[:, :] = out_ref[:, :] * scale


def _sc_gather(table, gidx):
    rows, width = table.shape
    (nidx,) = gidx.shape
    info = plsc.get_sparse_core_info()
    nw = info.num_cores * info.num_subcores
    b_per_w = nidx // nw
    chunk = 256
    mesh = plsc.VectorSubcoreMesh(core_axis_name="c", subcore_axis_name="s")

    @functools.partial(
        pl.kernel, mesh=mesh,
        out_type=jax.ShapeDtypeStruct((nidx, width), jnp.float32),
        scratch_types=[
            pltpu.VMEM((chunk,), jnp.int32),
            pltpu.VMEM((chunk, width), jnp.float32),
            pltpu.SemaphoreType.DMA,
        ],
    )
    def k(table_hbm, idx_hbm, out_hbm, idx_v, rows_v, sem):
        wid = lax.axis_index("s") * info.num_cores + lax.axis_index("c")
        base = wid * b_per_w

        @pl.loop(0, b_per_w // chunk)
        def _(c):
            off = base + c * chunk
            pltpu.sync_copy(idx_hbm.at[pl.ds(off, chunk)], idx_v)
            pltpu.async_copy(table_hbm.at[idx_v], rows_v, sem).wait()
            pltpu.sync_copy(rows_v, out_hbm.at[pl.ds(off, chunk)])

    return k(table, gidx)


def kernel(yp_white_pts, yp_pial_pts, yt_white_pts, yt_pial_pts,
           yp_white_idx, yt_white_idx, yp_pial_idx, yt_pial_idx):
    B, N, _ = yp_white_pts.shape
    nb = N // _QB

    P = jnp.stack([yp_white_pts, yp_pial_pts, yt_white_pts, yt_pial_pts])
    ones = jnp.ones(P.shape[:-1] + (1,), jnp.float32)
    P16 = jnp.concatenate(
        [P, jnp.zeros(P.shape[:-1] + (_PAD - 4,), jnp.float32), ones],
        axis=-1)
    PT = jnp.swapaxes(P16, 2, 3)

    sw = lambda d: (d // 2) * 2 + (1 - d % 2)

    th16 = pl.pallas_call(
        _th_kernel,
        grid=(4, B, nb),
        in_specs=[
            pl.BlockSpec((1, 1, _QB, _PAD), lambda d, b, i: (d, b, i, 0)),
            pl.BlockSpec((1, 1, _PAD, N), lambda d, b, i: (sw(d), b, 0, 0)),
            pl.BlockSpec((1, 1, N, _PAD), lambda d, b, i: (sw(d), b, 0, 0)),
        ],
        out_specs=pl.BlockSpec((1, 1, _QB, _W), lambda d, b, i: (d, b, i, 0)),
        out_shape=jax.ShapeDtypeStruct((4, B, N, _W), jnp.float32),
    )(P16, PT, P16)

    I = jnp.stack([yp_white_idx, yp_pial_idx, yt_white_idx, yt_pial_idx])
    I = I.astype(jnp.int32)
    parts = []
    for p in range(2):
        for s in range(2):
            f = p + 2 * (1 - s)
            for b in range(B):
                parts.append((f * B + b) * N + I[p + 2 * s, b])
    gidx = jnp.concatenate(parts)

    lg = _sc_gather(th16.reshape(4 * B * N, _W), gidx)

    lg = lg.reshape(2, 2, B, N, _W)
    loss = pl.pallas_call(
        functools.partial(_reduce_kernel, nbatch=B, scale=0.25 / (B * N)),
        grid=(2, 2, B),
        in_specs=[
            pl.BlockSpec((1, 1, N, _W),
                         lambda p, s, b: (p + 2 * s, b, 0, 0)),
            pl.BlockSpec((1, 1, 1, N, _W),
                         lambda p, s, b: (p, s, b, 0, 0)),
        ],
        out_specs=pl.BlockSpec((1, 1), lambda p, s, b: (0, 0)),
        out_shape=jax.ShapeDtypeStruct((1, 1), jnp.float32),
    )(th16, lg)

    return loss.reshape(())
